# baseline (speedup 1.0000x reference)
import jax
import jax.numpy as jnp
from jax import lax
from jax.experimental import pallas as pl
from jax.experimental.pallas import tpu as pltpu
from jax.experimental.pallas import tpu_sc as plsc

_B, _N = 4, 4096
_TQ = 1024


def _nn_icp_body(lastT_ref, nowv_ref, rotaT_ref, trans3_ref, out_ref,
                 kd_ref, kr_ref):
    b = pl.program_id(0)

    @pl.when(b == 0)
    def _init():
        out_ref[...] = jnp.zeros((1, 1), jnp.float32)

    vl = lastT_ref[0, 0:3, :]
    nl = lastT_ref[0, 3:6, :]
    kd_ref[0:3, :] = -2.0 * vl
    kd_ref[3:4, :] = jnp.sum(vl * vl, axis=0, keepdims=True)
    kr_ref[0:3, :] = nl
    kr_ref[3:4, :] = -jnp.sum(vl * nl, axis=0, keepdims=True)
    kd = kd_ref[...]
    kr = kr_ref[...].astype(jnp.bfloat16)

    acc = jnp.zeros((), jnp.float32)
    for q in range(_N // _TQ):
        vm = nowv_ref[0, q * _TQ:(q + 1) * _TQ, :]
        p = jnp.dot(vm, rotaT_ref[0], preferred_element_type=jnp.float32)
        p = p + trans3_ref[0]
        paug = jnp.concatenate([p, jnp.ones((_TQ, 1), jnp.float32)], axis=1)

        dmat = jnp.dot(paug, kd, preferred_element_type=jnp.float32)
        rmat = jnp.dot(paug.astype(jnp.bfloat16), kr,
                       preferred_element_type=jnp.float32)
        mrun = jnp.full((_TQ, 128), jnp.inf, jnp.float32)
        rrun = jnp.zeros((_TQ, 128), jnp.float32)
        for c in range(0, _N, 128):
            dc = dmat[:, c:c + 128]
            rc = rmat[:, c:c + 128]
            mask = dc < mrun
            rrun = jnp.where(mask, rc, rrun)
            mrun = jnp.minimum(mrun, dc)
        mf = jnp.min(mrun, axis=1, keepdims=True)
        r = jnp.sum(jnp.where(mrun == mf, rrun, 0.0), axis=1)
        acc += jnp.sum(jnp.abs(r))
    out_ref[...] += acc.reshape(1, 1)


def _prloss_sc_body(x_hbm, lo_hbm, hi_hbm, w0_hbm, w1_hbm, beta_hbm, out_hbm,
                    x_v, lo_v, hi_v, w0_v, w1_v, beta_v, o_v):
    cid = lax.axis_index("c")
    sid = lax.axis_index("s")

    @pl.when((cid == 0) & (sid == 0))
    def _():
        pltpu.sync_copy(x_hbm, x_v)
        pltpu.sync_copy(lo_hbm, lo_v)
        pltpu.sync_copy(hi_hbm, hi_v)
        pltpu.sync_copy(w0_hbm, w0_v)
        pltpu.sync_copy(w1_hbm, w1_v)
        pltpu.sync_copy(beta_hbm, beta_v)
        beta = beta_v[...]
        acc = jnp.zeros((16,), jnp.float32)
        for i in range(4):
            x = x_v[pl.ds(16 * i, 16)]
            lo = lo_v[pl.ds(16 * i, 16)]
            hi = hi_v[pl.ds(16 * i, 16)]
            w = w0_v[pl.ds(16 * i, 16)] + beta * w1_v[pl.ds(16 * i, 16)]
            d = x - jnp.minimum(jnp.maximum(x, lo), hi)
            acc = acc + w * d * d
        o_v[...] = acc
        pltpu.sync_copy(o_v, out_hbm)


def kernel(last_lossalldata, now_lossalldata, quat, trans, sx, sq, beta,
           bindex, needgtloss, rotainput):
    lastT = jnp.transpose(last_lossalldata, (0, 2, 1))
    nowv = now_lossalldata[:, :, :3]
    rotaT = jnp.transpose(quat, (0, 2, 1))
    trans3 = trans[:, None, :]

    out = pl.pallas_call(
        _nn_icp_body,
        grid=(_B,),
        in_specs=[
            pl.BlockSpec((1, 6, _N), lambda b: (b, 0, 0)),
            pl.BlockSpec((1, _N, 3), lambda b: (b, 0, 0)),
            pl.BlockSpec((1, 3, 3), lambda b: (b, 0, 0)),
            pl.BlockSpec((1, 1, 3), lambda b: (b, 0, 0)),
        ],
        out_specs=pl.BlockSpec((1, 1), lambda b: (0, 0)),
        out_shape=jax.ShapeDtypeStruct((1, 1), jnp.float32),
        scratch_shapes=[
            pltpu.VMEM((4, _N), jnp.float32),
            pltpu.VMEM((4, _N), jnp.float32),
        ],
        compiler_params=pltpu.CompilerParams(
            dimension_semantics=("arbitrary",)),
    )(lastT, nowv, rotaT, trans3)

    z4 = jnp.zeros((4,), jnp.float32)
    z8 = jnp.zeros((8,), jnp.float32)
    xcat = jnp.concatenate([
        trans.reshape(-1), z4,
        quat[:, :2, :].reshape(-1), z8,
        quat[:, 2, :].reshape(-1), z4,
    ])
    lo = jnp.concatenate([jnp.full((16,), -10.0), jnp.full((48,), -15.0)])
    hi = -lo
    w0 = jnp.concatenate([jnp.full((12,), 1.0 / 12), jnp.zeros((52,))])
    w1 = jnp.concatenate([jnp.zeros((16,)), jnp.full((24,), 1.0 / 24),
                          z8, jnp.full((12,), 1.0 / 12), z4])
    betav = jnp.full((16,), beta[0], jnp.float32)

    prk = pl.kernel(
        _prloss_sc_body,
        mesh=plsc.VectorSubcoreMesh(core_axis_name="c", subcore_axis_name="s"),
        out_type=jax.ShapeDtypeStruct((16,), jnp.float32),
        scratch_types=[
            pltpu.VMEM((64,), jnp.float32),
            pltpu.VMEM((64,), jnp.float32),
            pltpu.VMEM((64,), jnp.float32),
            pltpu.VMEM((64,), jnp.float32),
            pltpu.VMEM((64,), jnp.float32),
            pltpu.VMEM((16,), jnp.float32),
            pltpu.VMEM((16,), jnp.float32),
        ],
    )
    pr = prk(xcat, lo, hi, w0, w1, betav)
    return out[0, 0] + jnp.sum(pr)


# speedup vs baseline: 1.3078x; 1.3078x over previous
import jax
import jax.numpy as jnp
from jax.experimental import pallas as pl
from jax.experimental.pallas import tpu as pltpu

_B, _N = 4, 4096
_TQ = 1024


def _nn_icp_body(lastT_ref, nowv_ref, rotaT_ref, trans3_ref, quat_ref,
                 transf_ref, beta_ref, out_ref, kd_ref, kr_ref):
    b = pl.program_id(0)

    @pl.when(b == 0)
    def _init():
        quat = quat_ref[...]
        tr = transf_ref[...]
        beta = beta_ref[0, 0]
        dx = tr - jnp.clip(tr, -10.0, 10.0)
        loss_x = jnp.sum(dx * dx) * (1.0 / (_B * 3))
        dq1 = quat[:, :2, :] - jnp.clip(quat[:, :2, :], -15.0, 15.0)
        loss_q1 = jnp.sum(dq1 * dq1) * (1.0 / (_B * 2 * 3))
        dq2 = quat[:, 2, :] - jnp.clip(quat[:, 2, :], -15.0, 15.0)
        loss_q2 = jnp.sum(dq2 * dq2) * (1.0 / (_B * 3))
        out_ref[...] = (loss_x + (loss_q1 + loss_q2) * beta).reshape(1, 1)

    vl = lastT_ref[0, 0:3, :]
    nl = lastT_ref[0, 3:6, :]
    kd_ref[0:3, :] = -2.0 * vl
    kd_ref[3:4, :] = jnp.sum(vl * vl, axis=0, keepdims=True)
    kr_ref[0:3, :] = nl
    kr_ref[3:4, :] = -jnp.sum(vl * nl, axis=0, keepdims=True)
    kd = kd_ref[...]
    kr = kr_ref[...].astype(jnp.bfloat16)

    acc = jnp.zeros((), jnp.float32)
    for q in range(_N // _TQ):
        vm = nowv_ref[0, q * _TQ:(q + 1) * _TQ, :]
        p = jnp.dot(vm, rotaT_ref[0], preferred_element_type=jnp.float32)
        p = p + trans3_ref[0]
        paug = jnp.concatenate([p, jnp.ones((_TQ, 1), jnp.float32)], axis=1)

        dmat = jnp.dot(paug, kd, preferred_element_type=jnp.float32)
        rmat = jnp.dot(paug.astype(jnp.bfloat16), kr,
                       preferred_element_type=jnp.float32)
        mrun = jnp.full((_TQ, 128), jnp.inf, jnp.float32)
        rrun = jnp.zeros((_TQ, 128), jnp.float32)
        for c in range(0, _N, 128):
            dc = dmat[:, c:c + 128]
            rc = rmat[:, c:c + 128]
            mask = dc < mrun
            rrun = jnp.where(mask, rc, rrun)
            mrun = jnp.minimum(mrun, dc)
        mf = jnp.min(mrun, axis=1, keepdims=True)
        r = jnp.sum(jnp.where(mrun == mf, rrun, 0.0), axis=1)
        acc += jnp.sum(jnp.abs(r))
    out_ref[...] += acc.reshape(1, 1)


def kernel(last_lossalldata, now_lossalldata, quat, trans, sx, sq, beta,
           bindex, needgtloss, rotainput):
    lastT = jnp.transpose(last_lossalldata, (0, 2, 1))
    nowv = now_lossalldata[:, :, :3]
    rotaT = jnp.transpose(quat, (0, 2, 1))
    trans3 = trans[:, None, :]
    beta2 = beta.reshape(1, 1)

    out = pl.pallas_call(
        _nn_icp_body,
        grid=(_B,),
        in_specs=[
            pl.BlockSpec((1, 6, _N), lambda b: (b, 0, 0)),
            pl.BlockSpec((1, _N, 3), lambda b: (b, 0, 0)),
            pl.BlockSpec((1, 3, 3), lambda b: (b, 0, 0)),
            pl.BlockSpec((1, 1, 3), lambda b: (b, 0, 0)),
            pl.BlockSpec((_B, 3, 3), lambda b: (0, 0, 0)),
            pl.BlockSpec((_B, 3), lambda b: (0, 0)),
            pl.BlockSpec((1, 1), lambda b: (0, 0)),
        ],
        out_specs=pl.BlockSpec((1, 1), lambda b: (0, 0)),
        out_shape=jax.ShapeDtypeStruct((1, 1), jnp.float32),
        scratch_shapes=[
            pltpu.VMEM((4, _N), jnp.float32),
            pltpu.VMEM((4, _N), jnp.float32),
        ],
        compiler_params=pltpu.CompilerParams(
            dimension_semantics=("arbitrary",)),
    )(lastT, nowv, rotaT, trans3, quat, trans, beta2)
    return out[0, 0]
